# baseline (speedup 1.0000x reference)
import functools

import jax
import jax.numpy as jnp
from jax import lax
from jax.experimental import pallas as pl
from jax.experimental.pallas import tpu as pltpu
from jax.experimental.pallas import tpu_sc as plsc

VOCAB = 100000
EMBED_DIM = 64
BATCH = 4096
HIST = 50

NC = 2
NS = 16
NW = NC * NS

F_PER_W = EMBED_DIM // NW
N_VEC = BATCH // 16


def _make_kernel():
    mesh = plsc.VectorSubcoreMesh(core_axis_name="c", subcore_axis_name="s")

    @functools.partial(
        pl.kernel,
        out_type=jax.ShapeDtypeStruct((HIST, 8, 32, 8, 128), jnp.float32),
        mesh=mesh,
        compiler_params=pltpu.CompilerParams(
            use_tc_tiling_on_sc=False, needs_layout_passes=False
        ),
        scratch_types=[
            pltpu.VMEM((VOCAB,), jnp.float32),
            pltpu.VMEM((BATCH,), jnp.int32),
            pltpu.VMEM((32, 1, 128), jnp.float32),
            pltpu.SemaphoreType.DMA,
        ],
    )
    def emb_kernel(xt_hbm, tabt_hbm, out_hbm, frow_v, idx_v, vals_v, wsem):
        wid = lax.axis_index("c") * NS + lax.axis_index("s")

        for fk in range(F_PER_W):
            f = wid * F_PER_W + fk
            fo = f // 8
            fi = f % 8
            pltpu.sync_copy(tabt_hbm.at[f], frow_v)

            def per_h(h, _):
                pltpu.sync_copy(xt_hbm.at[h], idx_v)

                def per_vec(k, _):
                    idx = idx_v[pl.ds(k * 16, 16)]
                    vals_v[k // 8, 0, pl.ds((k % 8) * 16, 16)] = (
                        plsc.load_gather(frow_v, [idx])
                    )
                    return 0

                lax.fori_loop(0, N_VEC, per_vec, 0)
                pltpu.async_copy(
                    vals_v,
                    out_hbm.at[h, fo, :, pl.ds(fi, 1), :],
                    wsem,
                ).wait()
                return 0

            lax.fori_loop(0, HIST, per_h, 0)

    return emb_kernel


_emb_kernel = _make_kernel()


@jax.jit
def kernel(x, embed_weight):
    out5 = _emb_kernel(x.T.astype(jnp.int32), embed_weight.T)
    return out5.transpose(2, 4, 0, 1, 3).reshape(BATCH, HIST, EMBED_DIM)


# speedup vs baseline: 1.7861x; 1.7861x over previous
import functools

import jax
import jax.numpy as jnp
from jax import lax
from jax.experimental import pallas as pl
from jax.experimental.pallas import tpu as pltpu
from jax.experimental.pallas import tpu_sc as plsc

VOCAB = 100000
EMBED_DIM = 64
BATCH = 4096
HIST = 50

NC = 2
NS = 16
NW = NC * NS

F_PER_W = EMBED_DIM // NW
N_VEC = BATCH // 16


def _make_kernel():
    mesh = plsc.VectorSubcoreMesh(core_axis_name="c", subcore_axis_name="s")

    @functools.partial(
        pl.kernel,
        out_type=jax.ShapeDtypeStruct((HIST, 8, 32, 8, 128), jnp.float32),
        mesh=mesh,
        compiler_params=pltpu.CompilerParams(
            use_tc_tiling_on_sc=False, needs_layout_passes=False
        ),
        scratch_types=[
            pltpu.VMEM((VOCAB,), jnp.float32),
            pltpu.VMEM((2, BATCH), jnp.int32),
            pltpu.VMEM((2, 32, 1, 128), jnp.float32),
            [pltpu.SemaphoreType.DMA] * 2,
            [pltpu.SemaphoreType.DMA] * 2,
        ],
    )
    def emb_kernel(xt_hbm, tabt_hbm, out_hbm, frow_v, idx_v, vals_v, isem, wsem):
        wid = lax.axis_index("c") * NS + lax.axis_index("s")

        def idx_args(h, par):
            return (xt_hbm.at[h], idx_v.at[par], isem[par])

        for fk in range(F_PER_W):
            f = wid * F_PER_W + fk
            fo = f // 8
            fi = f % 8

            def write_args(h, par):
                return (
                    vals_v.at[par],
                    out_hbm.at[h, fo, :, pl.ds(fi, 1), :],
                    wsem[par],
                )

            pltpu.async_copy(*idx_args(0, 0))
            pltpu.sync_copy(tabt_hbm.at[f], frow_v)

            def per_pair(t, _):
                h0 = 2 * t
                for par in range(2):
                    h = h0 + par
                    pltpu.make_async_copy(*idx_args(h, par)).wait()

                    @pl.when(h + 1 < HIST)
                    def _():
                        pltpu.async_copy(*idx_args(h + 1, 1 - par))

                    @pl.when(h >= 2)
                    def _():
                        pltpu.make_async_copy(*write_args(h - 2, par)).wait()

                    for bo in range(32):
                        for q in range(8):
                            idx = idx_v[par, pl.ds(bo * 128 + q * 16, 16)]
                            vals_v[par, bo, 0, pl.ds(q * 16, 16)] = (
                                plsc.load_gather(frow_v, [idx])
                            )
                    pltpu.async_copy(*write_args(h, par))
                return 0

            lax.fori_loop(0, HIST // 2, per_pair, 0)
            for par in range(2):
                pltpu.make_async_copy(*write_args(HIST - 2 + par, par)).wait()

    return emb_kernel


_emb_kernel = _make_kernel()


@jax.jit
def kernel(x, embed_weight):
    out5 = _emb_kernel(x.T.astype(jnp.int32), embed_weight.T)
    return out5.transpose(2, 4, 0, 1, 3).reshape(BATCH, HIST, EMBED_DIM)


# speedup vs baseline: 2.2847x; 1.2791x over previous
import functools

import jax
import jax.numpy as jnp
from jax import lax
from jax.experimental import pallas as pl
from jax.experimental.pallas import tpu as pltpu
from jax.experimental.pallas import tpu_sc as plsc

VOCAB = 100000
EMBED_DIM = 64
BATCH = 4096
HIST = 50

NC = 2
NS = 16
NW = NC * NS

F_PER_W = EMBED_DIM // NW
N_VEC = BATCH // 16


def _make_kernel():
    mesh = plsc.VectorSubcoreMesh(core_axis_name="c", subcore_axis_name="s")

    @functools.partial(
        pl.kernel,
        out_type=jax.ShapeDtypeStruct((HIST, 8, 32, 8, 128), jnp.float32),
        mesh=mesh,
        compiler_params=pltpu.CompilerParams(
            use_tc_tiling_on_sc=False, needs_layout_passes=False
        ),
        scratch_types=[
            pltpu.VMEM((VOCAB,), jnp.float32),
            pltpu.VMEM((2, BATCH), jnp.int32),
            pltpu.VMEM((2, 32, 1, 128), jnp.float32),
            [pltpu.SemaphoreType.DMA] * 2,
            [pltpu.SemaphoreType.DMA] * 2,
        ],
    )
    def emb_kernel(xt_hbm, tabt_hbm, out_hbm, frow_v, idx_v, vals_v, isem, wsem):
        wid = lax.axis_index("c") * NS + lax.axis_index("s")

        def idx_args(h, par):
            return (xt_hbm.at[h], idx_v.at[par], isem[par])

        for fk in range(F_PER_W):
            f = wid * F_PER_W + fk
            fo = f // 8
            fi = f % 8

            def write_args(h, par):
                return (
                    vals_v.at[par],
                    out_hbm.at[h, fo, :, pl.ds(fi, 1), :],
                    wsem[par],
                )

            pltpu.async_copy(*idx_args(0, 0))
            pltpu.sync_copy(tabt_hbm.at[f], frow_v)

            def per_pair(t, _):
                h0 = 2 * t
                for par in range(2):
                    h = h0 + par
                    pltpu.make_async_copy(*idx_args(h, par)).wait()

                    @pl.when(h + 1 < HIST)
                    def _():
                        pltpu.async_copy(*idx_args(h + 1, 1 - par))

                    @pl.when(h >= 2)
                    def _():
                        pltpu.make_async_copy(*write_args(h - 2, par)).wait()

                    @functools.partial(
                        plsc.parallel_loop, 0, N_VEC, unroll=8
                    )
                    def _(k):
                        idx = idx_v[par, pl.ds(k * 16, 16)]
                        vals_v[par, k // 8, 0, pl.ds((k % 8) * 16, 16)] = (
                            plsc.load_gather(frow_v, [idx])
                        )
                    pltpu.async_copy(*write_args(h, par))
                return 0

            lax.fori_loop(0, HIST // 2, per_pair, 0)
            for par in range(2):
                pltpu.make_async_copy(*write_args(HIST - 2 + par, par)).wait()

    return emb_kernel


_emb_kernel = _make_kernel()


@jax.jit
def kernel(x, embed_weight):
    out5 = _emb_kernel(x.T.astype(jnp.int32), embed_weight.T)
    return out5.transpose(2, 4, 0, 1, 3).reshape(BATCH, HIST, EMBED_DIM)
